# baseline (speedup 1.0000x reference)
import functools

import jax
import jax.numpy as jnp
from jax import lax
from jax.experimental import pallas as pl
from jax.experimental.pallas import tpu as pltpu
from jax.experimental.pallas import tpu_sc as plsc

NC, NS, LANES = 2, 16, 16
NW = NC * NS


def _tc1_body(x_ref, w_ref, amat_ref, comb_ref, gmax_ref):
    y = jnp.dot(x_ref[...], w_ref[...], preferred_element_type=jnp.float32)
    s = jnp.dot(y, amat_ref[...], preferred_element_type=jnp.float32)
    comb_ref[...] = jnp.concatenate(
        [y, s, jnp.zeros((y.shape[0], 8), jnp.float32)], axis=1)
    m = jnp.max(s, axis=0)
    m = jnp.where(m > 0.0, m, 0.2 * m)
    row = jnp.concatenate([m, jnp.full((8,), 1e30, jnp.float32)])
    gmax_ref[...] = jnp.broadcast_to(row[None, :], (8, 16))


def _tc1(x, W1, amat):
    return pl.pallas_call(
        _tc1_body,
        out_shape=(
            jax.ShapeDtypeStruct((x.shape[0], 144), jnp.float32),
            jax.ShapeDtypeStruct((8, 16), jnp.float32),
        ),
    )(x, W1, amat)


E1 = 320000
EPW1 = E1 // NW
K1 = 64
ACC1_R = 2176
RPS = ACC1_R // NS
C1 = 144
NB1 = 3
CPAD1 = EPW1 + NB1 * K1


def _bcast(p, k):
    idx = jnp.full((LANES, 1), k, jnp.int32)
    dn = lax.GatherDimensionNumbers(
        offset_dims=(), collapsed_slice_dims=(0,), start_index_map=(0,))
    return lax.gather(p, idx, dn, slice_sizes=(1,),
                      mode=lax.GatherScatterMode.PROMISE_IN_BOUNDS)


def _zero_acc(zbuf, acc_sh, s, cols):
    for k in range(cols // LANES):
        zbuf[0, pl.ds(k * LANES, LANES)] = jnp.zeros((LANES,), jnp.float32)
    for r in range(1, 8):
        for k in range(cols // LANES):
            zbuf[r, pl.ds(k * LANES, LANES)] = zbuf[0, pl.ds(k * LANES, LANES)]

    def cpy(i, _):
        pltpu.sync_copy(zbuf, acc_sh.at[pl.ds(s * RPS + i * 8, 8)])
        return 0
    lax.fori_loop(0, RPS // 8, cpy, 0)


def _sc1_body(src_hbm, dst_hbm, nid_hbm, comb_hbm, gmax_hbm, out_hbm,
              nid_v, srcf, dstf, csrc, cdst, gidx0, gidx1, gidx2,
              dsti0, dsti1, dsti2, rows0, rows1, rows2,
              outb0, outb1, outb2, pscr, gvec, zbuf, acc_sh,
              gsem0, gsem1, gsem2, ssem0, ssem1, ssem2):
    c = lax.axis_index("c")
    s = lax.axis_index("s")
    wid = s * NC + c

    pltpu.sync_copy(nid_hbm, nid_v)
    pltpu.sync_copy(gmax_hbm.at[0], gvec)
    pltpu.sync_copy(src_hbm.at[pl.ds(wid * EPW1, EPW1)], srcf)
    pltpu.sync_copy(dst_hbm.at[pl.ds(wid * EPW1, EPW1)], dstf)

    _zero_acc(zbuf, acc_sh, s, C1)
    plsc.subcore_barrier()

    def pf(i, _):
        csrc[pl.ds(i * LANES, LANES)] = jnp.zeros((LANES,), jnp.int32)
        cdst[pl.ds(i * LANES, LANES)] = jnp.full((LANES,), 2048, jnp.int32)
        return 0
    lax.fori_loop(0, CPAD1 // LANES, pf, 0)

    def cp(i, n):
        svs = [srcf[pl.ds((i * 4 + u) * LANES, LANES)] for u in range(4)]
        dvs = [dstf[pl.ds((i * 4 + u) * LANES, LANES)] for u in range(4)]
        ms = [dv < 2048 for dv in dvs]
        pss = [plsc.cumsum(jnp.where(m, 1, 0).astype(jnp.int32))
               for m in ms]
        cnts = [plsc.all_reduce_population_count(m) for m in ms]
        for u in range(4):
            idx = n + pss[u] - 1
            plsc.store_scatter(csrc, [idx], svs[u], mask=ms[u])
            plsc.store_scatter(cdst, [idx], dvs[u], mask=ms[u])
            n = n + cnts[u]
        return n
    nvec = lax.fori_loop(0, EPW1 // (4 * LANES), cp,
                         jnp.zeros((LANES,), jnp.int32))
    nn = nvec[0]
    nouter = (nn + NB1 * K1 - 1) // (NB1 * K1)

    g16 = gvec[...]
    gidxs = (gidx0, gidx1, gidx2)
    dstis = (dsti0, dsti1, dsti2)
    rowss = (rows0, rows1, rows2)
    outbs = (outb0, outb1, outb2)
    gsems = (gsem0, gsem1, gsem2)
    ssems = (ssem0, ssem1, ssem2)

    def chunk(oi, _):
        base = oi * (NB1 * K1)
        descs = []
        for b in range(NB1):
            @pl.when(oi > 0)
            def _():
                pltpu.make_async_copy(
                    outbs[b], acc_sh.at[dstis[b]], ssems[b]).wait()
            bb = base + b * K1
            for t in range(K1 // LANES):
                sv = csrc[pl.ds(bb + t * LANES, LANES)]
                gidxs[b][pl.ds(t * LANES, LANES)] = (
                    plsc.load_gather(nid_v, [sv]))
                dstis[b][pl.ds(t * LANES, LANES)] = (
                    cdst[pl.ds(bb + t * LANES, LANES)])
            descs.append(
                pltpu.async_copy(comb_hbm.at[gidxs[b]], rowss[b], gsems[b]))
        for b in range(NB1):
            descs[b].wait()
            rows = rowss[b]
            outb = outbs[b]

            def edge(jj, _):
                js = [jj * 4 + u for u in range(4)]
                ps = []
                for j in js:
                    a = rows[j, pl.ds(128, LANES)]
                    a = jnp.where(a > 0.0, a, a * 0.2)
                    ps.append(jnp.exp(a - g16))
                for j, p in zip(js, ps):
                    xs = [rows[j, pl.ds(k * LANES, LANES)] for k in range(8)]
                    ws = [_bcast(p, k) for k in range(8)]
                    for k in range(8):
                        outb[j, pl.ds(k * LANES, LANES)] = xs[k] * ws[k]
                    outb[j, pl.ds(128, LANES)] = p
                return 0
            lax.fori_loop(0, K1 // 4, edge, 0)
            pltpu.async_copy(outb, acc_sh.at[dstis[b]], ssems[b], add=True)
        return 0
    lax.fori_loop(0, nouter, chunk, 0)

    @pl.when(nouter > 0)
    def _():
        for b in range(NB1):
            pltpu.make_async_copy(
                outbs[b], acc_sh.at[dstis[b]], ssems[b]).wait()

    plsc.subcore_barrier()
    pltpu.sync_copy(acc_sh.at[pl.ds(s * RPS, RPS)],
                    out_hbm.at[c, pl.ds(s * RPS, RPS)])


def _sc1(src, dst, n_id, comb, gmaxrow):
    mesh = plsc.VectorSubcoreMesh(core_axis_name="c", subcore_axis_name="s")
    f = pl.kernel(
        _sc1_body,
        out_type=jax.ShapeDtypeStruct((NC, ACC1_R, C1), jnp.float32),
        mesh=mesh,
        compiler_params=pltpu.CompilerParams(
            needs_layout_passes=False, use_tc_tiling_on_sc=False),
        scratch_types=[
            pltpu.VMEM((10000,), jnp.int32),
            pltpu.VMEM((EPW1,), jnp.int32),
            pltpu.VMEM((EPW1,), jnp.int32),
            pltpu.VMEM((CPAD1,), jnp.int32),
            pltpu.VMEM((CPAD1,), jnp.int32),
            pltpu.VMEM((K1,), jnp.int32),
            pltpu.VMEM((K1,), jnp.int32),
            pltpu.VMEM((K1,), jnp.int32),
            pltpu.VMEM((K1,), jnp.int32),
            pltpu.VMEM((K1,), jnp.int32),
            pltpu.VMEM((K1,), jnp.int32),
            pltpu.VMEM((K1, C1), jnp.float32),
            pltpu.VMEM((K1, C1), jnp.float32),
            pltpu.VMEM((K1, C1), jnp.float32),
            pltpu.VMEM((K1, C1), jnp.float32),
            pltpu.VMEM((K1, C1), jnp.float32),
            pltpu.VMEM((K1, C1), jnp.float32),
            pltpu.VMEM((3 * LANES,), jnp.float32),
            pltpu.VMEM((LANES,), jnp.float32),
            pltpu.VMEM((8, C1), jnp.float32),
            pltpu.VMEM_SHARED((ACC1_R, C1), jnp.float32),
            pltpu.SemaphoreType.DMA,
            pltpu.SemaphoreType.DMA,
            pltpu.SemaphoreType.DMA,
            pltpu.SemaphoreType.DMA,
            pltpu.SemaphoreType.DMA,
            pltpu.SemaphoreType.DMA,
        ],
    )
    return f(src, dst, n_id, comb, gmaxrow)


def _tc2_body(acc_ref, w2_ref, att2_ref, b1_ref, rep_ref, comb2_ref,
              gvec2_ref):
    A = acc_ref[0, :2048, :] + acc_ref[1, :2048, :]
    msg = A[:, :128]
    den = A[:, 128:136]
    den_rep = jnp.dot(den, rep_ref[...], preferred_element_type=jnp.float32)
    h1 = msg / (den_rep + 1e-30) + b1_ref[...]
    h1 = jnp.where(h1 > 0.0, h1, jnp.exp(h1) - 1.0)
    xl2 = jnp.dot(h1, w2_ref[...], preferred_element_type=jnp.float32)
    s2 = jnp.dot(xl2, att2_ref[...], preferred_element_type=jnp.float32)
    comb2_ref[...] = jnp.concatenate(
        [xl2, s2, jnp.zeros((2048, 15), jnp.float32)], axis=1)
    g2 = jnp.max(s2)
    g2 = jnp.where(g2 > 0.0, g2, 0.2 * g2)
    row = jnp.concatenate([g2[None], jnp.full((15,), 1e30, jnp.float32)])
    gvec2_ref[...] = jnp.broadcast_to(row[None, :], (8, 16))


def _tc2(acc1, W2, att2v, b1m, repm):
    return pl.pallas_call(
        _tc2_body,
        out_shape=(
            jax.ShapeDtypeStruct((2048, 80), jnp.float32),
            jax.ShapeDtypeStruct((8, 16), jnp.float32),
        ),
    )(acc1, W2, att2v, b1m, repm)


E2 = 65536
EPW2 = E2 // NW
K2 = 128
NO2 = EPW2 // (2 * K2)
C2 = 80
ACC2_R = 2176


def _sc2_body(src_hbm, dst_hbm, comb_hbm, gmax_hbm, out_hbm,
              srcb0, srcb1, dsti0, dsti1, rows0, rows1, outb0, outb1,
              pscr, gvec, zbuf, acc_sh, gsem0, gsem1, ssem0, ssem1):
    c = lax.axis_index("c")
    s = lax.axis_index("s")
    wid = s * NC + c

    pltpu.sync_copy(gmax_hbm.at[0], gvec)
    _zero_acc(zbuf, acc_sh, s, C2)
    plsc.subcore_barrier()

    g16 = gvec[...]
    srcbs = (srcb0, srcb1)
    dstis = (dsti0, dsti1)
    rowss = (rows0, rows1)
    outbs = (outb0, outb1)
    gsems = (gsem0, gsem1)
    ssems = (ssem0, ssem1)

    def chunk(oi, _):
        base = wid * EPW2 + oi * (2 * K2)
        descs = []
        for b in range(2):
            @pl.when(oi > 0)
            def _():
                pltpu.make_async_copy(
                    outbs[b], acc_sh.at[dstis[b]], ssems[b]).wait()
            bb = base + b * K2
            pltpu.sync_copy(src_hbm.at[pl.ds(bb, K2)], srcbs[b])
            pltpu.sync_copy(dst_hbm.at[pl.ds(bb, K2)], dstis[b])
            descs.append(
                pltpu.async_copy(comb_hbm.at[srcbs[b]], rowss[b], gsems[b]))
        for b in range(2):
            descs[b].wait()
            rows = rowss[b]
            outb = outbs[b]

            def edge(jj, _):
                js = [jj * 4 + u for u in range(4)]
                ps = []
                for j in js:
                    a = rows[j, pl.ds(64, LANES)]
                    a = jnp.where(a > 0.0, a, a * 0.2)
                    ps.append(jnp.exp(a - g16))
                for j, p in zip(js, ps):
                    xs = [rows[j, pl.ds(k * LANES, LANES)] for k in range(4)]
                    w = _bcast(p, 0)
                    for k in range(4):
                        outb[j, pl.ds(k * LANES, LANES)] = xs[k] * w
                    outb[j, pl.ds(64, LANES)] = p
                return 0
            lax.fori_loop(0, K2 // 4, edge, 0)
            pltpu.async_copy(outb, acc_sh.at[dstis[b]], ssems[b], add=True)
        return 0
    lax.fori_loop(0, NO2, chunk, 0)

    for b in range(2):
        pltpu.make_async_copy(outbs[b], acc_sh.at[dstis[b]], ssems[b]).wait()

    plsc.subcore_barrier()
    pltpu.sync_copy(acc_sh.at[pl.ds(s * RPS, RPS)],
                    out_hbm.at[c, pl.ds(s * RPS, RPS)])


def _sc2(src, dst, comb2, gvec2):
    mesh = plsc.VectorSubcoreMesh(core_axis_name="c", subcore_axis_name="s")
    f = pl.kernel(
        _sc2_body,
        out_type=jax.ShapeDtypeStruct((NC, ACC2_R, C2), jnp.float32),
        mesh=mesh,
        compiler_params=pltpu.CompilerParams(
            needs_layout_passes=False, use_tc_tiling_on_sc=False),
        scratch_types=[
            pltpu.VMEM((K2,), jnp.int32),
            pltpu.VMEM((K2,), jnp.int32),
            pltpu.VMEM((K2,), jnp.int32),
            pltpu.VMEM((K2,), jnp.int32),
            pltpu.VMEM((K2, C2), jnp.float32),
            pltpu.VMEM((K2, C2), jnp.float32),
            pltpu.VMEM((K2, C2), jnp.float32),
            pltpu.VMEM((K2, C2), jnp.float32),
            pltpu.VMEM((3 * LANES,), jnp.float32),
            pltpu.VMEM((LANES,), jnp.float32),
            pltpu.VMEM((8, C2), jnp.float32),
            pltpu.VMEM_SHARED((ACC2_R, C2), jnp.float32),
            pltpu.SemaphoreType.DMA,
            pltpu.SemaphoreType.DMA,
            pltpu.SemaphoreType.DMA,
            pltpu.SemaphoreType.DMA,
        ],
    )
    return f(src, dst, comb2, gvec2)


def _tc3_body(acc_ref, b2_ref, out_ref):
    A = acc_ref[0, :2048, :] + acc_ref[1, :2048, :]
    msg = A[:, :64]
    den = A[:, 64:65]
    o = msg / (den + 1e-30) + b2_ref[...]
    m = jnp.max(o, axis=1, keepdims=True)
    z = o - m
    lse = jnp.log(jnp.sum(jnp.exp(z), axis=1, keepdims=True))
    out_ref[...] = z - lse


def _tc3(acc2, b2m):
    return pl.pallas_call(
        _tc3_body,
        out_shape=jax.ShapeDtypeStruct((2048, 64), jnp.float32),
    )(acc2, b2m)


def kernel(x, n_id, edge_index1, edge_index2, num_dst1, num_dst2,
           W1, att1, b1, W2, att2, b2):
    heads, hid = 8, 16
    att1r = att1.reshape(heads, hid)
    amat = jnp.zeros((128, heads), jnp.float32)
    for h in range(heads):
        amat = amat.at[h * hid:(h + 1) * hid, h].set(att1r[h])
    repm = jnp.zeros((heads, 128), jnp.float32)
    for h in range(heads):
        repm = repm.at[h, h * hid:(h + 1) * hid].set(1.0)

    comb, gmaxrow = _tc1(x, W1, amat)
    src1 = edge_index1[0].astype(jnp.int32)
    dst1 = edge_index1[1].astype(jnp.int32)
    acc1 = _sc1(src1, dst1, n_id.astype(jnp.int32), comb, gmaxrow)

    comb2, gvec2 = _tc2(acc1, W2, att2.reshape(64, 1), b1.reshape(1, 128),
                        repm)
    src2 = edge_index2[0].astype(jnp.int32)
    dst2 = edge_index2[1].astype(jnp.int32)
    acc2 = _sc2(src2, dst2, comb2, gvec2)

    return _tc3(acc2, b2.reshape(1, 64))


# speedup vs baseline: 1.0011x; 1.0011x over previous
import functools

import jax
import jax.numpy as jnp
from jax import lax
from jax.experimental import pallas as pl
from jax.experimental.pallas import tpu as pltpu
from jax.experimental.pallas import tpu_sc as plsc

NC, NS, LANES = 2, 16, 16
NW = NC * NS


def _tc1_body(x_ref, w_ref, amat_ref, perm_ref, comb_ref, gmax_ref):
    y = jnp.dot(x_ref[...], w_ref[...], preferred_element_type=jnp.float32)
    s = jnp.dot(y, amat_ref[...], preferred_element_type=jnp.float32)
    hi = s.astype(jnp.bfloat16).astype(jnp.float32)
    lo = s - hi
    inter = jnp.dot(jnp.concatenate([hi, lo], axis=1), perm_ref[...],
                    preferred_element_type=jnp.float32)
    comb_ref[...] = jnp.concatenate(
        [y, inter, jnp.zeros((y.shape[0], 16), jnp.float32)],
        axis=1).astype(jnp.bfloat16)
    m = jnp.max(s, axis=0)
    m = jnp.where(m > 0.0, m, 0.2 * m)
    row = jnp.concatenate([m, jnp.full((8,), 1e30, jnp.float32)])
    gmax_ref[...] = jnp.broadcast_to(row[None, :], (8, 16))


def _tc1(x, W1, amat, perm):
    return pl.pallas_call(
        _tc1_body,
        out_shape=(
            jax.ShapeDtypeStruct((x.shape[0], 160), jnp.bfloat16),
            jax.ShapeDtypeStruct((8, 16), jnp.float32),
        ),
    )(x, W1, amat, perm)


E1 = 320000
EPW1 = E1 // NW
K1 = 64
ACC1_R = 2176
RPS = ACC1_R // NS
C1 = 144
NB1 = 3
CPAD1 = EPW1 + NB1 * K1


def _bcast(p, k):
    idx = jnp.full((LANES, 1), k, jnp.int32)
    dn = lax.GatherDimensionNumbers(
        offset_dims=(), collapsed_slice_dims=(0,), start_index_map=(0,))
    return lax.gather(p, idx, dn, slice_sizes=(1,),
                      mode=lax.GatherScatterMode.PROMISE_IN_BOUNDS)


def _bcast2(p, h0, h1):
    lane = lax.iota(jnp.int32, LANES)
    idx = jnp.where(lane < 8, h0, h1).reshape(LANES, 1)
    dn = lax.GatherDimensionNumbers(
        offset_dims=(), collapsed_slice_dims=(0,), start_index_map=(0,))
    return lax.gather(p, idx, dn, slice_sizes=(1,),
                      mode=lax.GatherScatterMode.PROMISE_IN_BOUNDS)


def _evi(k):
    return lax.iota(jnp.int32, LANES) * 2 + (32 * k)


def _odi(k):
    return lax.iota(jnp.int32, LANES) * 2 + (32 * k + 1)


def _zero_acc(zbuf, acc_sh, s, cols):
    for k in range(cols // LANES):
        zbuf[0, pl.ds(k * LANES, LANES)] = jnp.zeros((LANES,), jnp.float32)
    for r in range(1, 8):
        for k in range(cols // LANES):
            zbuf[r, pl.ds(k * LANES, LANES)] = zbuf[0, pl.ds(k * LANES, LANES)]

    def cpy(i, _):
        pltpu.sync_copy(zbuf, acc_sh.at[pl.ds(s * RPS + i * 8, 8)])
        return 0
    lax.fori_loop(0, RPS // 8, cpy, 0)


def _sc1_body(src_hbm, dst_hbm, nid_hbm, comb_hbm, gmax_hbm, out_hbm,
              nid_v, srcf, dstf, csrc, cdst, gidx0, gidx1, gidx2,
              dsti0, dsti1, dsti2, rows0, rows1, rows2,
              outb0, outb1, outb2, pscr, gvec, zbuf, acc_sh,
              gsem0, gsem1, gsem2, ssem0, ssem1, ssem2):
    c = lax.axis_index("c")
    s = lax.axis_index("s")
    wid = s * NC + c

    pltpu.sync_copy(nid_hbm, nid_v)
    pltpu.sync_copy(gmax_hbm.at[0], gvec)
    pltpu.sync_copy(src_hbm.at[pl.ds(wid * EPW1, EPW1)], srcf)
    pltpu.sync_copy(dst_hbm.at[pl.ds(wid * EPW1, EPW1)], dstf)

    _zero_acc(zbuf, acc_sh, s, C1)
    plsc.subcore_barrier()

    def pf(i, _):
        csrc[pl.ds(i * LANES, LANES)] = jnp.zeros((LANES,), jnp.int32)
        cdst[pl.ds(i * LANES, LANES)] = jnp.full((LANES,), 2048, jnp.int32)
        return 0
    lax.fori_loop(0, CPAD1 // LANES, pf, 0)

    def cp(i, n):
        svs = [srcf[pl.ds((i * 4 + u) * LANES, LANES)] for u in range(4)]
        dvs = [dstf[pl.ds((i * 4 + u) * LANES, LANES)] for u in range(4)]
        ms = [dv < 2048 for dv in dvs]
        pss = [plsc.cumsum(jnp.where(m, 1, 0).astype(jnp.int32))
               for m in ms]
        cnts = [plsc.all_reduce_population_count(m) for m in ms]
        for u in range(4):
            idx = n + pss[u] - 1
            plsc.store_scatter(csrc, [idx], svs[u], mask=ms[u])
            plsc.store_scatter(cdst, [idx], dvs[u], mask=ms[u])
            n = n + cnts[u]
        return n
    nvec = lax.fori_loop(0, EPW1 // (4 * LANES), cp,
                         jnp.zeros((LANES,), jnp.int32))
    nn = nvec[0]
    nouter = (nn + NB1 * K1 - 1) // (NB1 * K1)

    g16 = gvec[...]
    gidxs = (gidx0, gidx1, gidx2)
    dstis = (dsti0, dsti1, dsti2)
    rowss = (rows0, rows1, rows2)
    outbs = (outb0, outb1, outb2)
    gsems = (gsem0, gsem1, gsem2)
    ssems = (ssem0, ssem1, ssem2)

    def chunk(oi, _):
        base = oi * (NB1 * K1)
        descs = []
        for b in range(NB1):
            @pl.when(oi > 0)
            def _():
                pltpu.make_async_copy(
                    outbs[b], acc_sh.at[dstis[b]], ssems[b]).wait()
            bb = base + b * K1
            for t in range(K1 // LANES):
                sv = csrc[pl.ds(bb + t * LANES, LANES)]
                gidxs[b][pl.ds(t * LANES, LANES)] = (
                    plsc.load_gather(nid_v, [sv]))
                dstis[b][pl.ds(t * LANES, LANES)] = (
                    cdst[pl.ds(bb + t * LANES, LANES)])
            descs.append(
                pltpu.async_copy(comb_hbm.at[gidxs[b]], rowss[b], gsems[b]))
        for b in range(NB1):
            descs[b].wait()
            rows = rowss[b]
            outb = outbs[b]

            def edge(jj, _):
                js = [jj * 4 + u for u in range(4)]
                ps = []
                for j in js:
                    la = rows[j, pl.ds(128, 2 * LANES)]
                    hi, lo = plsc.unpack(
                        la, format=plsc.PackFormat.INTERLEAVED)
                    a = hi + lo
                    a = jnp.where(a > 0.0, a, a * 0.2)
                    ps.append(jnp.exp(a - g16))
                for j, p in zip(js, ps):
                    jv = jnp.full((LANES,), j, jnp.int32)
                    for k in range(4):
                        fb = rows[j, pl.ds(32 * k, 2 * LANES)]
                        ev, od = plsc.unpack(
                            fb, format=plsc.PackFormat.INTERLEAVED)
                        w2 = _bcast2(p, 2 * k, 2 * k + 1)
                        plsc.store_scatter(outb, [jv, _evi(k)], ev * w2)
                        plsc.store_scatter(outb, [jv, _odi(k)], od * w2)
                    outb[j, pl.ds(128, LANES)] = p
                return 0
            lax.fori_loop(0, K1 // 4, edge, 0)
            pltpu.async_copy(outb, acc_sh.at[dstis[b]], ssems[b], add=True)
        return 0
    lax.fori_loop(0, nouter, chunk, 0)

    @pl.when(nouter > 0)
    def _():
        for b in range(NB1):
            pltpu.make_async_copy(
                outbs[b], acc_sh.at[dstis[b]], ssems[b]).wait()

    plsc.subcore_barrier()
    pltpu.sync_copy(acc_sh.at[pl.ds(s * RPS, RPS)],
                    out_hbm.at[c, pl.ds(s * RPS, RPS)])


def _sc1(src, dst, n_id, comb, gmaxrow):
    mesh = plsc.VectorSubcoreMesh(core_axis_name="c", subcore_axis_name="s")
    f = pl.kernel(
        _sc1_body,
        out_type=jax.ShapeDtypeStruct((NC, ACC1_R, C1), jnp.float32),
        mesh=mesh,
        compiler_params=pltpu.CompilerParams(
            needs_layout_passes=False, use_tc_tiling_on_sc=False),
        scratch_types=[
            pltpu.VMEM((10000,), jnp.int32),
            pltpu.VMEM((EPW1,), jnp.int32),
            pltpu.VMEM((EPW1,), jnp.int32),
            pltpu.VMEM((CPAD1,), jnp.int32),
            pltpu.VMEM((CPAD1,), jnp.int32),
            pltpu.VMEM((K1,), jnp.int32),
            pltpu.VMEM((K1,), jnp.int32),
            pltpu.VMEM((K1,), jnp.int32),
            pltpu.VMEM((K1,), jnp.int32),
            pltpu.VMEM((K1,), jnp.int32),
            pltpu.VMEM((K1,), jnp.int32),
            pltpu.VMEM((K1, 160), jnp.bfloat16),
            pltpu.VMEM((K1, 160), jnp.bfloat16),
            pltpu.VMEM((K1, 160), jnp.bfloat16),
            pltpu.VMEM((K1, C1), jnp.float32),
            pltpu.VMEM((K1, C1), jnp.float32),
            pltpu.VMEM((K1, C1), jnp.float32),
            pltpu.VMEM((3 * LANES,), jnp.float32),
            pltpu.VMEM((LANES,), jnp.float32),
            pltpu.VMEM((8, C1), jnp.float32),
            pltpu.VMEM_SHARED((ACC1_R, C1), jnp.float32),
            pltpu.SemaphoreType.DMA,
            pltpu.SemaphoreType.DMA,
            pltpu.SemaphoreType.DMA,
            pltpu.SemaphoreType.DMA,
            pltpu.SemaphoreType.DMA,
            pltpu.SemaphoreType.DMA,
        ],
    )
    return f(src, dst, n_id, comb, gmaxrow)


def _tc2_body(acc_ref, w2_ref, att2_ref, b1_ref, rep_ref, comb2_ref,
              gvec2_ref):
    A = acc_ref[0, :2048, :] + acc_ref[1, :2048, :]
    msg = A[:, :128]
    den = A[:, 128:136]
    den_rep = jnp.dot(den, rep_ref[...], preferred_element_type=jnp.float32)
    h1 = msg / (den_rep + 1e-30) + b1_ref[...]
    h1 = jnp.where(h1 > 0.0, h1, jnp.exp(h1) - 1.0)
    xl2 = jnp.dot(h1, w2_ref[...], preferred_element_type=jnp.float32)
    s2 = jnp.dot(xl2, att2_ref[...], preferred_element_type=jnp.float32)
    comb2_ref[...] = jnp.concatenate(
        [xl2, s2, jnp.zeros((2048, 15), jnp.float32)], axis=1)
    g2 = jnp.max(s2)
    g2 = jnp.where(g2 > 0.0, g2, 0.2 * g2)
    row = jnp.concatenate([g2[None], jnp.full((15,), 1e30, jnp.float32)])
    gvec2_ref[...] = jnp.broadcast_to(row[None, :], (8, 16))


def _tc2(acc1, W2, att2v, b1m, repm):
    return pl.pallas_call(
        _tc2_body,
        out_shape=(
            jax.ShapeDtypeStruct((2048, 80), jnp.float32),
            jax.ShapeDtypeStruct((8, 16), jnp.float32),
        ),
    )(acc1, W2, att2v, b1m, repm)


E2 = 65536
EPW2 = E2 // NW
K2 = 128
NO2 = EPW2 // (2 * K2)
C2 = 80
ACC2_R = 2176


def _sc2_body(src_hbm, dst_hbm, comb_hbm, gmax_hbm, out_hbm,
              srcb0, srcb1, dsti0, dsti1, rows0, rows1, outb0, outb1,
              pscr, gvec, zbuf, acc_sh, gsem0, gsem1, ssem0, ssem1):
    c = lax.axis_index("c")
    s = lax.axis_index("s")
    wid = s * NC + c

    pltpu.sync_copy(gmax_hbm.at[0], gvec)
    _zero_acc(zbuf, acc_sh, s, C2)
    plsc.subcore_barrier()

    g16 = gvec[...]
    srcbs = (srcb0, srcb1)
    dstis = (dsti0, dsti1)
    rowss = (rows0, rows1)
    outbs = (outb0, outb1)
    gsems = (gsem0, gsem1)
    ssems = (ssem0, ssem1)

    def chunk(oi, _):
        base = wid * EPW2 + oi * (2 * K2)
        descs = []
        for b in range(2):
            @pl.when(oi > 0)
            def _():
                pltpu.make_async_copy(
                    outbs[b], acc_sh.at[dstis[b]], ssems[b]).wait()
            bb = base + b * K2
            pltpu.sync_copy(src_hbm.at[pl.ds(bb, K2)], srcbs[b])
            pltpu.sync_copy(dst_hbm.at[pl.ds(bb, K2)], dstis[b])
            descs.append(
                pltpu.async_copy(comb_hbm.at[srcbs[b]], rowss[b], gsems[b]))
        for b in range(2):
            descs[b].wait()
            rows = rowss[b]
            outb = outbs[b]

            def edge(jj, _):
                js = [jj * 4 + u for u in range(4)]
                ps = []
                for j in js:
                    a = rows[j, pl.ds(64, LANES)]
                    a = jnp.where(a > 0.0, a, a * 0.2)
                    ps.append(jnp.exp(a - g16))
                for j, p in zip(js, ps):
                    xs = [rows[j, pl.ds(k * LANES, LANES)] for k in range(4)]
                    w = _bcast(p, 0)
                    for k in range(4):
                        outb[j, pl.ds(k * LANES, LANES)] = xs[k] * w
                    outb[j, pl.ds(64, LANES)] = p
                return 0
            lax.fori_loop(0, K2 // 4, edge, 0)
            pltpu.async_copy(outb, acc_sh.at[dstis[b]], ssems[b], add=True)
        return 0
    lax.fori_loop(0, NO2, chunk, 0)

    for b in range(2):
        pltpu.make_async_copy(outbs[b], acc_sh.at[dstis[b]], ssems[b]).wait()

    plsc.subcore_barrier()
    pltpu.sync_copy(acc_sh.at[pl.ds(s * RPS, RPS)],
                    out_hbm.at[c, pl.ds(s * RPS, RPS)])


def _sc2(src, dst, comb2, gvec2):
    mesh = plsc.VectorSubcoreMesh(core_axis_name="c", subcore_axis_name="s")
    f = pl.kernel(
        _sc2_body,
        out_type=jax.ShapeDtypeStruct((NC, ACC2_R, C2), jnp.float32),
        mesh=mesh,
        compiler_params=pltpu.CompilerParams(
            needs_layout_passes=False, use_tc_tiling_on_sc=False),
        scratch_types=[
            pltpu.VMEM((K2,), jnp.int32),
            pltpu.VMEM((K2,), jnp.int32),
            pltpu.VMEM((K2,), jnp.int32),
            pltpu.VMEM((K2,), jnp.int32),
            pltpu.VMEM((K2, C2), jnp.float32),
            pltpu.VMEM((K2, C2), jnp.float32),
            pltpu.VMEM((K2, C2), jnp.float32),
            pltpu.VMEM((K2, C2), jnp.float32),
            pltpu.VMEM((3 * LANES,), jnp.float32),
            pltpu.VMEM((LANES,), jnp.float32),
            pltpu.VMEM((8, C2), jnp.float32),
            pltpu.VMEM_SHARED((ACC2_R, C2), jnp.float32),
            pltpu.SemaphoreType.DMA,
            pltpu.SemaphoreType.DMA,
            pltpu.SemaphoreType.DMA,
            pltpu.SemaphoreType.DMA,
        ],
    )
    return f(src, dst, comb2, gvec2)


def _tc3_body(acc_ref, b2_ref, out_ref):
    A = acc_ref[0, :2048, :] + acc_ref[1, :2048, :]
    msg = A[:, :64]
    den = A[:, 64:65]
    o = msg / (den + 1e-30) + b2_ref[...]
    m = jnp.max(o, axis=1, keepdims=True)
    z = o - m
    lse = jnp.log(jnp.sum(jnp.exp(z), axis=1, keepdims=True))
    out_ref[...] = z - lse


def _tc3(acc2, b2m):
    return pl.pallas_call(
        _tc3_body,
        out_shape=jax.ShapeDtypeStruct((2048, 64), jnp.float32),
    )(acc2, b2m)


def kernel(x, n_id, edge_index1, edge_index2, num_dst1, num_dst2,
           W1, att1, b1, W2, att2, b2):
    heads, hid = 8, 16
    att1r = att1.reshape(heads, hid)
    amat = jnp.zeros((128, heads), jnp.float32)
    for h in range(heads):
        amat = amat.at[h * hid:(h + 1) * hid, h].set(att1r[h])
    repm = jnp.zeros((heads, 128), jnp.float32)
    for h in range(heads):
        repm = repm.at[h, h * hid:(h + 1) * hid].set(1.0)

    perm = jnp.zeros((16, 16), jnp.float32)
    for h in range(8):
        perm = perm.at[h, 2 * h].set(1.0)
        perm = perm.at[8 + h, 2 * h + 1].set(1.0)

    comb, gmaxrow = _tc1(x, W1, amat, perm)
    src1 = edge_index1[0].astype(jnp.int32)
    dst1 = edge_index1[1].astype(jnp.int32)
    acc1 = _sc1(src1, dst1, n_id.astype(jnp.int32), comb, gmaxrow)

    comb2, gvec2 = _tc2(acc1, W2, att2.reshape(64, 1), b1.reshape(1, 128),
                        repm)
    src2 = edge_index2[0].astype(jnp.int32)
    dst2 = edge_index2[1].astype(jnp.int32)
    acc2 = _sc2(src2, dst2, comb2, gvec2)

    return _tc3(acc2, b2.reshape(1, 64))


# speedup vs baseline: 1.0091x; 1.0080x over previous
import functools

import jax
import jax.numpy as jnp
from jax import lax
from jax.experimental import pallas as pl
from jax.experimental.pallas import tpu as pltpu
from jax.experimental.pallas import tpu_sc as plsc

NC, NS, LANES = 2, 16, 16
NW = NC * NS


def _tc1_body(x_ref, w_ref, amat_ref, perm_ref, comb_ref, gmax_ref):
    y = jnp.dot(x_ref[...], w_ref[...], preferred_element_type=jnp.float32)
    s = jnp.dot(y, amat_ref[...], preferred_element_type=jnp.float32)
    hi = s.astype(jnp.bfloat16).astype(jnp.float32)
    lo = s - hi
    inter = jnp.dot(jnp.concatenate([hi, lo], axis=1), perm_ref[...],
                    preferred_element_type=jnp.float32)
    comb_ref[...] = jnp.concatenate(
        [y, inter, jnp.zeros((y.shape[0], 16), jnp.float32)],
        axis=1).astype(jnp.bfloat16)
    m = jnp.max(s, axis=0)
    m = jnp.where(m > 0.0, m, 0.2 * m)
    row = jnp.concatenate([m, jnp.full((8,), 1e30, jnp.float32)])
    gmax_ref[...] = jnp.broadcast_to(row[None, :], (8, 16))


def _tc1(x, W1, amat, perm):
    return pl.pallas_call(
        _tc1_body,
        out_shape=(
            jax.ShapeDtypeStruct((x.shape[0], 160), jnp.bfloat16),
            jax.ShapeDtypeStruct((8, 16), jnp.float32),
        ),
    )(x, W1, amat, perm)


E1 = 320000
EPW1 = E1 // NW
K1 = 64
ACC1_R = 2176
RPS = ACC1_R // NS
C1 = 144
NB1 = 3
CPAD1 = EPW1 + NB1 * K1


def _bcast(p, k):
    idx = jnp.full((LANES, 1), k, jnp.int32)
    dn = lax.GatherDimensionNumbers(
        offset_dims=(), collapsed_slice_dims=(0,), start_index_map=(0,))
    return lax.gather(p, idx, dn, slice_sizes=(1,),
                      mode=lax.GatherScatterMode.PROMISE_IN_BOUNDS)


def _bcast2(p, h0, h1):
    lane = lax.iota(jnp.int32, LANES)
    idx = jnp.where(lane < 8, h0, h1).reshape(LANES, 1)
    dn = lax.GatherDimensionNumbers(
        offset_dims=(), collapsed_slice_dims=(0,), start_index_map=(0,))
    return lax.gather(p, idx, dn, slice_sizes=(1,),
                      mode=lax.GatherScatterMode.PROMISE_IN_BOUNDS)


def _evi(k):
    return lax.iota(jnp.int32, LANES) * 2 + (32 * k)


def _odi(k):
    return lax.iota(jnp.int32, LANES) * 2 + (32 * k + 1)


def _zero_acc(zbuf, acc_sh, s, cols):
    for k in range(cols // LANES):
        zbuf[0, pl.ds(k * LANES, LANES)] = jnp.zeros((LANES,), jnp.float32)
    for r in range(1, 8):
        for k in range(cols // LANES):
            zbuf[r, pl.ds(k * LANES, LANES)] = zbuf[0, pl.ds(k * LANES, LANES)]

    def cpy(i, _):
        pltpu.sync_copy(zbuf, acc_sh.at[pl.ds(s * RPS + i * 8, 8)])
        return 0
    lax.fori_loop(0, RPS // 8, cpy, 0)


def _sc1_body(src_hbm, dst_hbm, nid_hbm, comb_hbm, gmax_hbm, out_hbm,
              nid_v, srcf, dstf, csrc, cdst, gidx0, gidx1, gidx2,
              dsti0, dsti1, dsti2, rows0, rows1, rows2,
              outb0, outb1, outb2, pscr, gvec, zbuf, acc_sh,
              gsem0, gsem1, gsem2, gsem3, gsem4, gsem5,
              ssem0, ssem1, ssem2):
    c = lax.axis_index("c")
    s = lax.axis_index("s")
    wid = s * NC + c

    pltpu.sync_copy(nid_hbm, nid_v)
    pltpu.sync_copy(gmax_hbm.at[0], gvec)
    pltpu.sync_copy(src_hbm.at[pl.ds(wid * EPW1, EPW1)], srcf)
    pltpu.sync_copy(dst_hbm.at[pl.ds(wid * EPW1, EPW1)], dstf)

    _zero_acc(zbuf, acc_sh, s, C1)
    plsc.subcore_barrier()

    def pf(i, _):
        csrc[pl.ds(i * LANES, LANES)] = jnp.zeros((LANES,), jnp.int32)
        cdst[pl.ds(i * LANES, LANES)] = jnp.full((LANES,), 2048, jnp.int32)
        return 0
    lax.fori_loop(0, CPAD1 // LANES, pf, 0)

    def cp(i, n):
        svs = [srcf[pl.ds((i * 4 + u) * LANES, LANES)] for u in range(4)]
        dvs = [dstf[pl.ds((i * 4 + u) * LANES, LANES)] for u in range(4)]
        ms = [dv < 2048 for dv in dvs]
        pss = [plsc.cumsum(jnp.where(m, 1, 0).astype(jnp.int32))
               for m in ms]
        cnts = [plsc.all_reduce_population_count(m) for m in ms]
        for u in range(4):
            idx = n + pss[u] - 1
            plsc.store_scatter(csrc, [idx], svs[u], mask=ms[u])
            plsc.store_scatter(cdst, [idx], dvs[u], mask=ms[u])
            n = n + cnts[u]
        return n
    nvec = lax.fori_loop(0, EPW1 // (4 * LANES), cp,
                         jnp.zeros((LANES,), jnp.int32))
    nn = nvec[0]
    nouter = (nn + NB1 * K1 - 1) // (NB1 * K1)

    g16 = gvec[...]
    gidxs = (gidx0, gidx1, gidx2)
    dstis = (dsti0, dsti1, dsti2)
    rowss = (rows0, rows1, rows2)
    outbs = (outb0, outb1, outb2)
    gsems = (gsem0, gsem1, gsem2)
    gsems2 = (gsem3, gsem4, gsem5)
    ssems = (ssem0, ssem1, ssem2)

    def chunk(oi, _):
        base = oi * (NB1 * K1)
        descs = []
        for b in range(NB1):
            @pl.when(oi > 0)
            def _():
                pltpu.make_async_copy(
                    outbs[b], acc_sh.at[dstis[b]], ssems[b]).wait()
            bb = base + b * K1
            for t in range(K1 // LANES):
                sv = csrc[pl.ds(bb + t * LANES, LANES)]
                gidxs[b][pl.ds(t * LANES, LANES)] = (
                    plsc.load_gather(nid_v, [sv]))
                dstis[b][pl.ds(t * LANES, LANES)] = (
                    cdst[pl.ds(bb + t * LANES, LANES)])
            descs.append((
                pltpu.async_copy(comb_hbm.at[gidxs[b].at[pl.ds(0, K1 // 2)]],
                                 rowss[b].at[pl.ds(0, K1 // 2)], gsems[b]),
                pltpu.async_copy(comb_hbm.at[gidxs[b].at[pl.ds(K1 // 2,
                                                               K1 // 2)]],
                                 rowss[b].at[pl.ds(K1 // 2, K1 // 2)],
                                 gsems2[b])))
        for b in range(NB1):
            descs[b][0].wait()
            descs[b][1].wait()
            rows = rowss[b]
            outb = outbs[b]

            def edge(jj, _):
                js = [jj * 4 + u for u in range(4)]
                ps = []
                for j in js:
                    la = rows[j, pl.ds(128, 2 * LANES)]
                    hi, lo = plsc.unpack(
                        la, format=plsc.PackFormat.INTERLEAVED)
                    a = hi + lo
                    a = jnp.where(a > 0.0, a, a * 0.2)
                    ps.append(jnp.exp(a - g16))
                for j, p in zip(js, ps):
                    jv = jnp.full((LANES,), j, jnp.int32)
                    for k in range(4):
                        fb = rows[j, pl.ds(32 * k, 2 * LANES)]
                        ev, od = plsc.unpack(
                            fb, format=plsc.PackFormat.INTERLEAVED)
                        w2 = _bcast2(p, 2 * k, 2 * k + 1)
                        plsc.store_scatter(outb, [jv, _evi(k)], ev * w2)
                        plsc.store_scatter(outb, [jv, _odi(k)], od * w2)
                    outb[j, pl.ds(128, LANES)] = p
                return 0
            lax.fori_loop(0, K1 // 4, edge, 0)
            pltpu.async_copy(outb, acc_sh.at[dstis[b]], ssems[b], add=True)
        return 0
    lax.fori_loop(0, nouter, chunk, 0)

    @pl.when(nouter > 0)
    def _():
        for b in range(NB1):
            pltpu.make_async_copy(
                outbs[b], acc_sh.at[dstis[b]], ssems[b]).wait()

    plsc.subcore_barrier()
    pltpu.sync_copy(acc_sh.at[pl.ds(s * RPS, RPS)],
                    out_hbm.at[c, pl.ds(s * RPS, RPS)])


def _sc1(src, dst, n_id, comb, gmaxrow):
    mesh = plsc.VectorSubcoreMesh(core_axis_name="c", subcore_axis_name="s")
    f = pl.kernel(
        _sc1_body,
        out_type=jax.ShapeDtypeStruct((NC, ACC1_R, C1), jnp.float32),
        mesh=mesh,
        compiler_params=pltpu.CompilerParams(
            needs_layout_passes=False, use_tc_tiling_on_sc=False),
        scratch_types=[
            pltpu.VMEM((10000,), jnp.int32),
            pltpu.VMEM((EPW1,), jnp.int32),
            pltpu.VMEM((EPW1,), jnp.int32),
            pltpu.VMEM((CPAD1,), jnp.int32),
            pltpu.VMEM((CPAD1,), jnp.int32),
            pltpu.VMEM((K1,), jnp.int32),
            pltpu.VMEM((K1,), jnp.int32),
            pltpu.VMEM((K1,), jnp.int32),
            pltpu.VMEM((K1,), jnp.int32),
            pltpu.VMEM((K1,), jnp.int32),
            pltpu.VMEM((K1,), jnp.int32),
            pltpu.VMEM((K1, 160), jnp.bfloat16),
            pltpu.VMEM((K1, 160), jnp.bfloat16),
            pltpu.VMEM((K1, 160), jnp.bfloat16),
            pltpu.VMEM((K1, C1), jnp.float32),
            pltpu.VMEM((K1, C1), jnp.float32),
            pltpu.VMEM((K1, C1), jnp.float32),
            pltpu.VMEM((3 * LANES,), jnp.float32),
            pltpu.VMEM((LANES,), jnp.float32),
            pltpu.VMEM((8, C1), jnp.float32),
            pltpu.VMEM_SHARED((ACC1_R, C1), jnp.float32),
            pltpu.SemaphoreType.DMA,
            pltpu.SemaphoreType.DMA,
            pltpu.SemaphoreType.DMA,
            pltpu.SemaphoreType.DMA,
            pltpu.SemaphoreType.DMA,
            pltpu.SemaphoreType.DMA,
            pltpu.SemaphoreType.DMA,
            pltpu.SemaphoreType.DMA,
            pltpu.SemaphoreType.DMA,
        ],
    )
    return f(src, dst, n_id, comb, gmaxrow)


def _tc2_body(acc_ref, w2_ref, att2_ref, b1_ref, rep_ref, comb2_ref,
              gvec2_ref):
    A = acc_ref[0, :2048, :] + acc_ref[1, :2048, :]
    msg = A[:, :128]
    den = A[:, 128:136]
    den_rep = jnp.dot(den, rep_ref[...], preferred_element_type=jnp.float32)
    h1 = msg / (den_rep + 1e-30) + b1_ref[...]
    h1 = jnp.where(h1 > 0.0, h1, jnp.exp(h1) - 1.0)
    xl2 = jnp.dot(h1, w2_ref[...], preferred_element_type=jnp.float32)
    s2 = jnp.dot(xl2, att2_ref[...], preferred_element_type=jnp.float32)
    comb2_ref[...] = jnp.concatenate(
        [xl2, s2, jnp.zeros((2048, 15), jnp.float32)], axis=1)
    g2 = jnp.max(s2)
    g2 = jnp.where(g2 > 0.0, g2, 0.2 * g2)
    row = jnp.concatenate([g2[None], jnp.full((15,), 1e30, jnp.float32)])
    gvec2_ref[...] = jnp.broadcast_to(row[None, :], (8, 16))


def _tc2(acc1, W2, att2v, b1m, repm):
    return pl.pallas_call(
        _tc2_body,
        out_shape=(
            jax.ShapeDtypeStruct((2048, 80), jnp.float32),
            jax.ShapeDtypeStruct((8, 16), jnp.float32),
        ),
    )(acc1, W2, att2v, b1m, repm)


E2 = 65536
EPW2 = E2 // NW
K2 = 128
NO2 = EPW2 // (2 * K2)
C2 = 80
ACC2_R = 2176


def _sc2_body(src_hbm, dst_hbm, comb_hbm, gmax_hbm, out_hbm,
              srcb0, srcb1, dsti0, dsti1, rows0, rows1, outb0, outb1,
              pscr, gvec, zbuf, acc_sh, gsem0, gsem1, ssem0, ssem1):
    c = lax.axis_index("c")
    s = lax.axis_index("s")
    wid = s * NC + c

    pltpu.sync_copy(gmax_hbm.at[0], gvec)
    _zero_acc(zbuf, acc_sh, s, C2)
    plsc.subcore_barrier()

    g16 = gvec[...]
    srcbs = (srcb0, srcb1)
    dstis = (dsti0, dsti1)
    rowss = (rows0, rows1)
    outbs = (outb0, outb1)
    gsems = (gsem0, gsem1)
    ssems = (ssem0, ssem1)

    def chunk(oi, _):
        base = wid * EPW2 + oi * (2 * K2)
        descs = []
        for b in range(2):
            @pl.when(oi > 0)
            def _():
                pltpu.make_async_copy(
                    outbs[b], acc_sh.at[dstis[b]], ssems[b]).wait()
            bb = base + b * K2
            pltpu.sync_copy(src_hbm.at[pl.ds(bb, K2)], srcbs[b])
            pltpu.sync_copy(dst_hbm.at[pl.ds(bb, K2)], dstis[b])
            descs.append(
                pltpu.async_copy(comb_hbm.at[srcbs[b]], rowss[b], gsems[b]))
        for b in range(2):
            descs[b].wait()
            rows = rowss[b]
            outb = outbs[b]

            def edge(jj, _):
                js = [jj * 4 + u for u in range(4)]
                ps = []
                for j in js:
                    a = rows[j, pl.ds(64, LANES)]
                    a = jnp.where(a > 0.0, a, a * 0.2)
                    ps.append(jnp.exp(a - g16))
                for j, p in zip(js, ps):
                    xs = [rows[j, pl.ds(k * LANES, LANES)] for k in range(4)]
                    w = _bcast(p, 0)
                    for k in range(4):
                        outb[j, pl.ds(k * LANES, LANES)] = xs[k] * w
                    outb[j, pl.ds(64, LANES)] = p
                return 0
            lax.fori_loop(0, K2 // 4, edge, 0)
            pltpu.async_copy(outb, acc_sh.at[dstis[b]], ssems[b], add=True)
        return 0
    lax.fori_loop(0, NO2, chunk, 0)

    for b in range(2):
        pltpu.make_async_copy(outbs[b], acc_sh.at[dstis[b]], ssems[b]).wait()

    plsc.subcore_barrier()
    pltpu.sync_copy(acc_sh.at[pl.ds(s * RPS, RPS)],
                    out_hbm.at[c, pl.ds(s * RPS, RPS)])


def _sc2(src, dst, comb2, gvec2):
    mesh = plsc.VectorSubcoreMesh(core_axis_name="c", subcore_axis_name="s")
    f = pl.kernel(
        _sc2_body,
        out_type=jax.ShapeDtypeStruct((NC, ACC2_R, C2), jnp.float32),
        mesh=mesh,
        compiler_params=pltpu.CompilerParams(
            needs_layout_passes=False, use_tc_tiling_on_sc=False),
        scratch_types=[
            pltpu.VMEM((K2,), jnp.int32),
            pltpu.VMEM((K2,), jnp.int32),
            pltpu.VMEM((K2,), jnp.int32),
            pltpu.VMEM((K2,), jnp.int32),
            pltpu.VMEM((K2, C2), jnp.float32),
            pltpu.VMEM((K2, C2), jnp.float32),
            pltpu.VMEM((K2, C2), jnp.float32),
            pltpu.VMEM((K2, C2), jnp.float32),
            pltpu.VMEM((3 * LANES,), jnp.float32),
            pltpu.VMEM((LANES,), jnp.float32),
            pltpu.VMEM((8, C2), jnp.float32),
            pltpu.VMEM_SHARED((ACC2_R, C2), jnp.float32),
            pltpu.SemaphoreType.DMA,
            pltpu.SemaphoreType.DMA,
            pltpu.SemaphoreType.DMA,
            pltpu.SemaphoreType.DMA,
        ],
    )
    return f(src, dst, comb2, gvec2)


def _tc3_body(acc_ref, b2_ref, out_ref):
    A = acc_ref[0, :2048, :] + acc_ref[1, :2048, :]
    msg = A[:, :64]
    den = A[:, 64:65]
    o = msg / (den + 1e-30) + b2_ref[...]
    m = jnp.max(o, axis=1, keepdims=True)
    z = o - m
    lse = jnp.log(jnp.sum(jnp.exp(z), axis=1, keepdims=True))
    out_ref[...] = z - lse


def _tc3(acc2, b2m):
    return pl.pallas_call(
        _tc3_body,
        out_shape=jax.ShapeDtypeStruct((2048, 64), jnp.float32),
    )(acc2, b2m)


def kernel(x, n_id, edge_index1, edge_index2, num_dst1, num_dst2,
           W1, att1, b1, W2, att2, b2):
    heads, hid = 8, 16
    att1r = att1.reshape(heads, hid)
    amat = jnp.zeros((128, heads), jnp.float32)
    for h in range(heads):
        amat = amat.at[h * hid:(h + 1) * hid, h].set(att1r[h])
    repm = jnp.zeros((heads, 128), jnp.float32)
    for h in range(heads):
        repm = repm.at[h, h * hid:(h + 1) * hid].set(1.0)

    perm = jnp.zeros((16, 16), jnp.float32)
    for h in range(8):
        perm = perm.at[h, 2 * h].set(1.0)
        perm = perm.at[8 + h, 2 * h + 1].set(1.0)

    comb, gmaxrow = _tc1(x, W1, amat, perm)
    src1 = edge_index1[0].astype(jnp.int32)
    dst1 = edge_index1[1].astype(jnp.int32)
    acc1 = _sc1(src1, dst1, n_id.astype(jnp.int32), comb, gmaxrow)

    comb2, gvec2 = _tc2(acc1, W2, att2.reshape(64, 1), b1.reshape(1, 128),
                        repm)
    src2 = edge_index2[0].astype(jnp.int32)
    dst2 = edge_index2[1].astype(jnp.int32)
    acc2 = _sc2(src2, dst2, comb2, gvec2)

    return _tc3(acc2, b2.reshape(1, 64))
